# baseline (speedup 1.0000x reference)
import functools

import numpy as np
import jax
import jax.numpy as jnp
from jax import lax
from jax.experimental import pallas as pl
from jax.experimental.pallas import tpu as pltpu
from jax.experimental.pallas import tpu_sc as plsc

_NEAR = 2.0
_FAR = 6.0
_T = 2 ** 19
_MASK = _T - 1
_P2 = int(np.int32(np.uint32(2654435761)))
_P3 = int(np.int32(np.uint32(805459861)))
_NC, _NS, _LANES = 2, 16, 16
_NW = _NC * _NS


def _res_list(L, base, mx):
    growth = np.exp((np.log(mx) - np.log(base)) / max(L - 1, 1))
    return [int(np.floor(base * growth ** l)) for l in range(L)]


def _sample_positions(n, key):
    eps = float(jnp.finfo(jnp.float32).eps)
    inter_len = eps + (1.0 - eps) / n
    pos = jnp.linspace(0.0, 1.0 - inter_len, n)
    max_jitter = (1.0 - inter_len) / (n - 1) - eps
    shift = jax.random.uniform(key, (n,)) * max_jitter
    return pos + shift


def _cumsum_last(x):
    n = x.shape[-1]
    k = 1
    while k < n:
        x = x + jnp.concatenate(
            [jnp.zeros_like(x[..., :k]), x[..., :-k]], axis=-1)
        k *= 2
    return x


def _softplus(x):
    return jnp.maximum(x, 0.0) + jnp.log1p(jnp.exp(-jnp.abs(x)))


def _contract_to_unit(x, y, z):
    r = jnp.sqrt(x * x + y * y + z * z)
    rs = jnp.maximum(r, 1e-9)
    scale = jnp.where(r <= 1.0, 1.0, (2.0 - 1.0 / rs) / rs)
    def enc(c):
        return jnp.clip((c * scale + 2.0) / 4.0, 0.0, 1.0 - 1e-6)
    return enc(x), enc(y), enc(z)


def _vol_weight(sigma, sd):
    td = _NEAR * (1.0 - sd) + _FAR * sd
    delta = td[:, 1:] - td[:, :-1]
    dd = sigma * delta
    alpha = 1.0 - jnp.exp(-dd)
    csum = _cumsum_last(dd[:, :-1])
    trans = jnp.exp(-jnp.concatenate(
        [jnp.zeros_like(dd[:, :1]), csum], axis=-1))
    return alpha * trans, td


def _invert_cdf_block(smp, sd, cw, n_in):
    R = sd.shape[0]
    big = jnp.float32(np.inf)
    fp0 = jnp.full((R, smp.shape[1]), -big)
    fp1 = jnp.full((R, smp.shape[1]), big)
    xp0 = jnp.full((R, smp.shape[1]), -big)
    xp1 = jnp.full((R, smp.shape[1]), big)
    for e in range(n_in + 1):
        if e == 0:
            cdf_e = jnp.zeros((R, 1), jnp.float32)
        elif e == n_in:
            cdf_e = jnp.ones((R, 1), jnp.float32)
        else:
            cdf_e = cw[:, e - 1:e]
        sd_e = sd[:, e:e + 1]
        mask = smp >= cdf_e
        fp0 = jnp.maximum(fp0, jnp.where(mask, sd_e, -big))
        fp1 = jnp.minimum(fp1, jnp.where(mask, big, sd_e))
        xp0 = jnp.maximum(xp0, jnp.where(mask, cdf_e, -big))
        xp1 = jnp.minimum(xp1, jnp.where(mask, big, cdf_e))
    raw = (smp - xp0) / (xp1 - xp0)
    off = jnp.clip(jnp.where(raw != raw, 0.0, raw), 0.0, 1.0)
    return fp0 + off * (fp1 - fp0)


def _resample_and_project(sd, weight, smp, ro, rdn, n_in):
    logits = jnp.where(sd[:, 1:] > sd[:, :-1],
                       jnp.log(jnp.maximum(weight, 1e-30)),
                       -jnp.float32(np.inf))
    m = jnp.max(logits, axis=-1, keepdims=True)
    une = jnp.exp(logits - m)
    wsm = une / jnp.sum(une, axis=-1, keepdims=True)
    cw = jnp.minimum(_cumsum_last(wsm[:, :-1]), 1.0)
    new = _invert_cdf_block(smp, sd, cw, n_in)
    mid = (new[:, 1:] + new[:, :-1]) / 2.0
    first = jnp.maximum(2.0 * new[:, :1] - mid[:, :1], 0.0)
    last = jnp.minimum(2.0 * new[:, -1:] - mid[:, -1:], 1.0)
    sd_next = jnp.concatenate([first, mid, last], axis=-1)
    t = _NEAR * (1.0 - new) + _FAR * new
    x = ro[:, 0:1] + rdn[:, 0:1] * t
    y = ro[:, 1:2] + rdn[:, 1:2] * t
    z = ro[:, 2:3] + rdn[:, 2:3] * t
    ex, ey, ez = _contract_to_unit(x, y, z)
    return sd_next, ex, ey, ez


def _prep0_body(ro_ref, rd_ref, t0_ref, x01_ref, rdn_ref):
    rd = rd_ref[...]
    nrm = jnp.sqrt(jnp.sum(rd * rd, axis=-1, keepdims=True))
    rdn = rd / jnp.maximum(nrm, 1e-9)
    rdn_ref[...] = rdn
    ro = ro_ref[...]
    t = t0_ref[...]
    x = ro[:, 0:1] + rdn[:, 0:1] * t
    y = ro[:, 1:2] + rdn[:, 1:2] * t
    z = ro[:, 2:3] + rdn[:, 2:3] * t
    ex, ey, ez = _contract_to_unit(x, y, z)
    x01_ref[0] = ex
    x01_ref[1] = ey
    x01_ref[2] = ez


def _prep0(rays_o, rays_d, t0, R=256):
    B = rays_o.shape[0]
    grid = (B // R,)
    return pl.pallas_call(
        _prep0_body,
        grid=grid,
        in_specs=[
            pl.BlockSpec((R, 3), lambda i: (i, 0)),
            pl.BlockSpec((R, 3), lambda i: (i, 0)),
            pl.BlockSpec((1, 64), lambda i: (0, 0)),
        ],
        out_specs=[
            pl.BlockSpec((3, R, 64), lambda i: (0, i, 0)),
            pl.BlockSpec((R, 3), lambda i: (i, 0)),
        ],
        out_shape=[
            jax.ShapeDtypeStruct((3, B, 64), jnp.float32),
            jax.ShapeDtypeStruct((B, 3), jnp.float32),
        ],
    )(rays_o, rays_d, t0.reshape(1, 64))


def _stage_body(n_in, n_out, R, K,
                feat_ref, sd_ref, ro_ref, rdn_ref, W1_ref, b1_ref,
                W2_ref, b2_ref, smp_ref, x01_ref, sdn_ref):
    h3 = jnp.zeros((R, n_in, 64), jnp.float32) + b1_ref[...][None]
    for k in range(K):
        h3 = h3 + feat_ref[k][:, :, None] * W1_ref[k][None, None, :]
    h3 = jnp.maximum(h3, 0.0)
    sig_lin = jnp.sum(h3 * W2_ref[...][None], axis=-1) + b2_ref[0, 0]
    sigma = _softplus(sig_lin)
    sd = sd_ref[...]
    weight, _ = _vol_weight(sigma, sd)
    sd_next, ex, ey, ez = _resample_and_project(
        sd, weight, smp_ref[...], ro_ref[...], rdn_ref[...], n_in)
    sdn_ref[...] = sd_next
    x01_ref[0] = ex
    x01_ref[1] = ey
    x01_ref[2] = ez


def _stage(feats, sd_prev, rays_o, rdn, W1, b1, W2, b2, smp, n_in, n_out,
           R=128):
    B = rays_o.shape[0]
    K = feats.shape[0]
    grid = (B // R,)
    body = functools.partial(_stage_body, n_in, n_out, R, K)
    return pl.pallas_call(
        body,
        grid=grid,
        in_specs=[
            pl.BlockSpec((K, R, n_in), lambda i: (0, i, 0)),
            pl.BlockSpec((R, n_in + 1), lambda i: (i, 0)),
            pl.BlockSpec((R, 3), lambda i: (i, 0)),
            pl.BlockSpec((R, 3), lambda i: (i, 0)),
            pl.BlockSpec((K, 64), lambda i: (0, 0)),
            pl.BlockSpec((1, 64), lambda i: (0, 0)),
            pl.BlockSpec((1, 64), lambda i: (0, 0)),
            pl.BlockSpec((1, 1), lambda i: (0, 0)),
            pl.BlockSpec((1, n_out), lambda i: (0, 0)),
        ],
        out_specs=[
            pl.BlockSpec((3, R, n_out), lambda i: (0, i, 0)),
            pl.BlockSpec((R, n_out + 1), lambda i: (i, 0)),
        ],
        out_shape=[
            jax.ShapeDtypeStruct((3, B, n_out), jnp.float32),
            jax.ShapeDtypeStruct((B, n_out + 1), jnp.float32),
        ],
    )(feats, sd_prev, rays_o, rdn, W1, b1.reshape(1, 64),
      W2.reshape(1, 64), b2.reshape(1, 1), smp.reshape(1, n_out))


def _render_body(R, K, feat_ref, sd_ref, rdn_ref, W1_ref, b1_ref, Wsig_ref,
                 bsig_ref, Wb_ref, W2a_ref, W2b_ref, b2_ref, Wrgb_ref,
                 brgb_ref, comp_ref, w_ref, op_ref, dep_ref):
    n = 32
    h3 = jnp.zeros((R, n, 64), jnp.float32) + b1_ref[...][None]
    for k in range(K):
        h3 = h3 + feat_ref[k][:, :, None] * W1_ref[k][None, None, :]
    h3 = jnp.maximum(h3, 0.0)
    h = h3.reshape(R * n, 64)
    sig_lin = jnp.sum(h3 * Wsig_ref[...][None], axis=-1) + bsig_ref[0, 0]
    sigma = _softplus(sig_lin)

    v = rdn_ref[...]
    embs = [v]
    for i in range(4):
        embs.append(jnp.sin(v * (2.0 ** i)))
        embs.append(jnp.cos(v * (2.0 ** i)))
    vemb = jnp.concatenate(embs, axis=-1)
    vw = jnp.dot(vemb, W2b_ref[...], preferred_element_type=jnp.float32)
    wbw = jnp.dot(Wb_ref[...], W2a_ref[...],
                  preferred_element_type=jnp.float32)
    h2l = jnp.dot(h, wbw, preferred_element_type=jnp.float32)
    h2 = jnp.maximum(
        h2l.reshape(R, n, 64) + vw[:, None, :] + b2_ref[...][None], 0.0)
    rgb_l = jnp.dot(h2.reshape(R * n, 64), Wrgb_ref[...],
                    preferred_element_type=jnp.float32) + brgb_ref[...]
    rgb = jax.nn.sigmoid(rgb_l).reshape(R, n, 3)

    sd = sd_ref[...]
    w, td = _vol_weight(sigma, sd)
    opacity = jnp.sum(w, axis=-1, keepdims=True)
    comp = jnp.sum(w[:, :, None] * rgb, axis=1) + (1.0 - opacity)
    tmid = (td[:, 1:] + td[:, :-1]) / 2.0
    depth = jnp.sum(w * tmid, axis=-1, keepdims=True)
    comp_ref[...] = comp
    w_ref[...] = w
    op_ref[...] = opacity
    dep_ref[...] = depth


def _render(feats, sd, rdn, p, R=128):
    B = sd.shape[0]
    n = 32
    K = feats.shape[0]
    grid = (B // R,)
    body = functools.partial(_render_body, R, K)
    return pl.pallas_call(
        body,
        grid=grid,
        in_specs=[
            pl.BlockSpec((K, R, n), lambda i: (0, i, 0)),
            pl.BlockSpec((R, n + 1), lambda i: (i, 0)),
            pl.BlockSpec((R, 3), lambda i: (i, 0)),
            pl.BlockSpec((K, 64), lambda i: (0, 0)),
            pl.BlockSpec((1, 64), lambda i: (0, 0)),
            pl.BlockSpec((1, 64), lambda i: (0, 0)),
            pl.BlockSpec((1, 1), lambda i: (0, 0)),
            pl.BlockSpec((64, 15), lambda i: (0, 0)),
            pl.BlockSpec((15, 64), lambda i: (0, 0)),
            pl.BlockSpec((27, 64), lambda i: (0, 0)),
            pl.BlockSpec((1, 64), lambda i: (0, 0)),
            pl.BlockSpec((64, 3), lambda i: (0, 0)),
            pl.BlockSpec((1, 3), lambda i: (0, 0)),
        ],
        out_specs=[
            pl.BlockSpec((R, 3), lambda i: (i, 0)),
            pl.BlockSpec((R, n), lambda i: (i, 0)),
            pl.BlockSpec((R, 1), lambda i: (i, 0)),
            pl.BlockSpec((R, 1), lambda i: (i, 0)),
        ],
        out_shape=[
            jax.ShapeDtypeStruct((B, 3), jnp.float32),
            jax.ShapeDtypeStruct((B, n), jnp.float32),
            jax.ShapeDtypeStruct((B, 1), jnp.float32),
            jax.ShapeDtypeStruct((B, 1), jnp.float32),
        ],
    )(feats, sd, rdn,
      p['rend_W1'], p['rend_b1'].reshape(1, 64),
      p['rend_Wsig'].reshape(1, 64), p['rend_bsig'].reshape(1, 1),
      p['rend_Wb'], p['rend_W2'][:15], p['rend_W2'][15:],
      p['rend_b2'].reshape(1, 64), p['rend_Wrgb'],
      p['rend_brgb'].reshape(1, 3))


def _make_encode(N, L, D, base, mx):
    res = _res_list(L, base, mx)
    C = 128
    npw = N // _NW
    nch = npw // C
    K = L * D
    mesh = plsc.VectorSubcoreMesh(core_axis_name="c", subcore_axis_name="s")
    nvec = C // _LANES

    def body(xs, ys, zs, tab, out, xv, yv, zv, fxv, fyv, fzv,
             hxv, hyv, hzv, idxv, wv, valv, featv, semg):
        wid = lax.axis_index("s") * _NC + lax.axis_index("c")
        base0 = wid * npw

        def chunk(ci, carry):
            pbase = base0 + ci * C
            pltpu.sync_copy(xs.at[pl.ds(pbase, C)], xv)
            pltpu.sync_copy(ys.at[pl.ds(pbase, C)], yv)
            pltpu.sync_copy(zs.at[pl.ds(pbase, C)], zv)
            for l in range(L):
                resf = jnp.float32(res[l])
                off = l * _T

                def prep(i, c2):
                    s = pl.ds(i * _LANES, _LANES)
                    px = xv[s] * resf
                    py = yv[s] * resf
                    pz = zv[s] * resf
                    ix = px.astype(jnp.int32)
                    iy = py.astype(jnp.int32)
                    iz = pz.astype(jnp.int32)
                    fxv[s] = px - ix.astype(jnp.float32)
                    fyv[s] = py - iy.astype(jnp.float32)
                    fzv[s] = pz - iz.astype(jnp.float32)
                    hxv[s] = ix
                    hyv[s] = iy * _P2
                    hzv[s] = iz * _P3
                    return c2
                lax.fori_loop(0, nvec, prep, 0)

                def fire(c, c2):
                    cx = lax.shift_right_logical(c, 2) & 1
                    cy = lax.shift_right_logical(c, 1) & 1
                    cz = c & 1
                    cxf = cx.astype(jnp.float32)
                    cyf = cy.astype(jnp.float32)
                    czf = cz.astype(jnp.float32)

                    def fv(i, c3):
                        s = pl.ds(i * _LANES, _LANES)
                        hx = hxv[s] + cx
                        hy = hyv[s] + cy * _P2
                        hz = hzv[s] + cz * _P3
                        hh = ((hx ^ hy) ^ hz) & _MASK
                        if D == 1:
                            idxv[c, s] = hh + off
                        else:
                            base_i = hh * D + off * D
                            for d in range(D):
                                idxv[c * D + d, s] = base_i + d
                        fx = fxv[s]
                        fy = fyv[s]
                        fz = fzv[s]
                        wx = cxf * fx + (1.0 - cxf) * (1.0 - fx)
                        wy = cyf * fy + (1.0 - cyf) * (1.0 - fy)
                        wz = czf * fz + (1.0 - czf) * (1.0 - fz)
                        wv[c, s] = wx * wy * wz
                        return c3
                    lax.fori_loop(0, nvec, fv, 0)
                    for d in range(D):
                        j = c * D + d
                        pltpu.async_copy(tab.at[idxv.at[j]], valv.at[j], semg)
                    return c2
                lax.fori_loop(0, 8, fire, 0)

                def drain(c, c2):
                    pltpu.make_async_copy(
                        tab.at[pl.ds(0, C)], valv.at[c], semg).wait()
                    return c2
                lax.fori_loop(0, 8 * D, drain, 0)

                def acc(i, c2):
                    s = pl.ds(i * _LANES, _LANES)
                    if D == 1:
                        def inner(c, a):
                            return a + wv[c, s] * valv[c, s]
                        a = lax.fori_loop(
                            0, 8, inner, jnp.zeros((16,), jnp.float32))
                        featv[l, s] = a
                    else:
                        def inner(c, a):
                            w = wv[c, s]
                            return (a[0] + w * valv[c * D, s],
                                    a[1] + w * valv[c * D + 1, s])
                        a = lax.fori_loop(
                            0, 8, inner,
                            (jnp.zeros((16,), jnp.float32),
                             jnp.zeros((16,), jnp.float32)))
                        featv[l * D, s] = a[0]
                        featv[l * D + 1, s] = a[1]
                    return c2
                lax.fori_loop(0, nvec, acc, 0)
            pltpu.sync_copy(featv, out.at[:, pl.ds(pbase, C)])
            return carry

        lax.fori_loop(0, nch, chunk, 0)

    kern = pl.kernel(
        body,
        out_type=jax.ShapeDtypeStruct((K, N), jnp.float32),
        mesh=mesh,
        scratch_types=[
            pltpu.VMEM((C,), jnp.float32),
            pltpu.VMEM((C,), jnp.float32),
            pltpu.VMEM((C,), jnp.float32),
            pltpu.VMEM((C,), jnp.float32),
            pltpu.VMEM((C,), jnp.float32),
            pltpu.VMEM((C,), jnp.float32),
            pltpu.VMEM((C,), jnp.int32),
            pltpu.VMEM((C,), jnp.int32),
            pltpu.VMEM((C,), jnp.int32),
            pltpu.VMEM((8 * D, C), jnp.int32),
            pltpu.VMEM((8, C), jnp.float32),
            pltpu.VMEM((8 * D, C), jnp.float32),
            pltpu.VMEM((K, C), jnp.float32),
            pltpu.SemaphoreType.DMA,
        ],
    )
    return kern


def _encode(x01, table, L, D, base, mx):
    B, n = x01.shape[1], x01.shape[2]
    N = B * n
    xs = x01[0].reshape(N)
    ys = x01[1].reshape(N)
    zs = x01[2].reshape(N)
    tab = table.reshape(L * _T * D)
    out = _make_encode(N, L, D, base, mx)(xs, ys, zs, tab)
    return out.reshape(L * D, B, n)


def kernel(rays_o, rays_d, params):
    B = rays_d.shape[0]

    s0 = _sample_positions(64, jax.random.key(100))
    s1 = _sample_positions(64, jax.random.key(101))
    s2 = _sample_positions(32, jax.random.key(999))

    new0 = s0
    mid0 = (new0[1:] + new0[:-1]) / 2.0
    first0 = jnp.maximum(2.0 * new0[:1] - mid0[:1], 0.0)
    last0 = jnp.minimum(2.0 * new0[-1:] - mid0[-1:], 1.0)
    sd0 = jnp.concatenate([first0, mid0, last0], axis=-1)
    t0 = _NEAR * (1.0 - new0) + _FAR * new0

    x01_0, rdn = _prep0(rays_o, rays_d, t0)

    feats0 = _encode(x01_0, params['prop_table_0'], 8, 1, 16, 512)
    x01_1, sd1 = _stage(
        feats0, jnp.broadcast_to(sd0, (B, 65)), rays_o, rdn,
        params['prop_W1_0'], params['prop_b1_0'],
        params['prop_W2_0'], params['prop_b2_0'], s1, 64, 64)

    feats1 = _encode(x01_1, params['prop_table_1'], 8, 1, 16, 512)
    x01_2, sd2 = _stage(
        feats1, sd1, rays_o, rdn,
        params['prop_W1_1'], params['prop_b1_1'],
        params['prop_W2_1'], params['prop_b2_1'], s2, 64, 32)

    feats2 = _encode(x01_2, params['render_table'], 16, 2, 16, 2048)
    comp, w, opacity, depth = _render(feats2, sd2, rdn, params)

    return (comp.reshape(B, 1, 1, 3), w.reshape(B, 1, 1, 32),
            opacity.reshape(B, 1, 1), depth.reshape(B, 1, 1))


# speedup vs baseline: 1.0343x; 1.0343x over previous
import functools

import numpy as np
import jax
import jax.numpy as jnp
from jax import lax
from jax.experimental import pallas as pl
from jax.experimental.pallas import tpu as pltpu
from jax.experimental.pallas import tpu_sc as plsc

_NEAR = 2.0
_FAR = 6.0
_T = 2 ** 19
_MASK = _T - 1
_P2 = int(np.int32(np.uint32(2654435761)))
_P3 = int(np.int32(np.uint32(805459861)))
_NC, _NS, _LANES = 2, 16, 16
_NW = _NC * _NS


def _res_list(L, base, mx):
    growth = np.exp((np.log(mx) - np.log(base)) / max(L - 1, 1))
    return [int(np.floor(base * growth ** l)) for l in range(L)]


def _sample_positions(n, key):
    eps = float(jnp.finfo(jnp.float32).eps)
    inter_len = eps + (1.0 - eps) / n
    pos = jnp.linspace(0.0, 1.0 - inter_len, n)
    max_jitter = (1.0 - inter_len) / (n - 1) - eps
    shift = jax.random.uniform(key, (n,)) * max_jitter
    return pos + shift


def _cumsum_last(x):
    n = x.shape[-1]
    k = 1
    while k < n:
        x = x + jnp.concatenate(
            [jnp.zeros_like(x[..., :k]), x[..., :-k]], axis=-1)
        k *= 2
    return x


def _softplus(x):
    return jnp.maximum(x, 0.0) + jnp.log1p(jnp.exp(-jnp.abs(x)))


def _contract_to_unit(x, y, z):
    r = jnp.sqrt(x * x + y * y + z * z)
    rs = jnp.maximum(r, 1e-9)
    scale = jnp.where(r <= 1.0, 1.0, (2.0 - 1.0 / rs) / rs)
    def enc(c):
        return jnp.clip((c * scale + 2.0) / 4.0, 0.0, 1.0 - 1e-6)
    return enc(x), enc(y), enc(z)


def _vol_weight(sigma, sd):
    td = _NEAR * (1.0 - sd) + _FAR * sd
    delta = td[:, 1:] - td[:, :-1]
    dd = sigma * delta
    alpha = 1.0 - jnp.exp(-dd)
    csum = _cumsum_last(dd[:, :-1])
    trans = jnp.exp(-jnp.concatenate(
        [jnp.zeros_like(dd[:, :1]), csum], axis=-1))
    return alpha * trans, td


def _invert_cdf_block(smp, sd, cw, n_in):
    R = sd.shape[0]
    big = jnp.float32(np.inf)
    fp0 = jnp.full((R, smp.shape[1]), -big)
    fp1 = jnp.full((R, smp.shape[1]), big)
    xp0 = jnp.full((R, smp.shape[1]), -big)
    xp1 = jnp.full((R, smp.shape[1]), big)
    for e in range(n_in + 1):
        if e == 0:
            cdf_e = jnp.zeros((R, 1), jnp.float32)
        elif e == n_in:
            cdf_e = jnp.ones((R, 1), jnp.float32)
        else:
            cdf_e = cw[:, e - 1:e]
        sd_e = sd[:, e:e + 1]
        mask = smp >= cdf_e
        fp0 = jnp.maximum(fp0, jnp.where(mask, sd_e, -big))
        fp1 = jnp.minimum(fp1, jnp.where(mask, big, sd_e))
        xp0 = jnp.maximum(xp0, jnp.where(mask, cdf_e, -big))
        xp1 = jnp.minimum(xp1, jnp.where(mask, big, cdf_e))
    raw = (smp - xp0) / (xp1 - xp0)
    off = jnp.clip(jnp.where(raw != raw, 0.0, raw), 0.0, 1.0)
    return fp0 + off * (fp1 - fp0)


def _resample_and_project(sd, weight, smp, ro, rdn, n_in):
    logits = jnp.where(sd[:, 1:] > sd[:, :-1],
                       jnp.log(jnp.maximum(weight, 1e-30)),
                       -jnp.float32(np.inf))
    m = jnp.max(logits, axis=-1, keepdims=True)
    une = jnp.exp(logits - m)
    wsm = une / jnp.sum(une, axis=-1, keepdims=True)
    cw = jnp.minimum(_cumsum_last(wsm[:, :-1]), 1.0)
    new = _invert_cdf_block(smp, sd, cw, n_in)
    mid = (new[:, 1:] + new[:, :-1]) / 2.0
    first = jnp.maximum(2.0 * new[:, :1] - mid[:, :1], 0.0)
    last = jnp.minimum(2.0 * new[:, -1:] - mid[:, -1:], 1.0)
    sd_next = jnp.concatenate([first, mid, last], axis=-1)
    t = _NEAR * (1.0 - new) + _FAR * new
    x = ro[:, 0:1] + rdn[:, 0:1] * t
    y = ro[:, 1:2] + rdn[:, 1:2] * t
    z = ro[:, 2:3] + rdn[:, 2:3] * t
    ex, ey, ez = _contract_to_unit(x, y, z)
    return sd_next, ex, ey, ez


def _prep0_body(ro_ref, rd_ref, t0_ref, x01_ref, rdn_ref):
    rd = rd_ref[...]
    nrm = jnp.sqrt(jnp.sum(rd * rd, axis=-1, keepdims=True))
    rdn = rd / jnp.maximum(nrm, 1e-9)
    rdn_ref[...] = rdn
    ro = ro_ref[...]
    t = t0_ref[...]
    x = ro[:, 0:1] + rdn[:, 0:1] * t
    y = ro[:, 1:2] + rdn[:, 1:2] * t
    z = ro[:, 2:3] + rdn[:, 2:3] * t
    ex, ey, ez = _contract_to_unit(x, y, z)
    pad = jnp.zeros((ex.shape[0], 128 - ex.shape[1]), jnp.float32)
    x01_ref[0] = jnp.concatenate([ex, pad], -1)
    x01_ref[1] = jnp.concatenate([ey, pad], -1)
    x01_ref[2] = jnp.concatenate([ez, pad], -1)


def _prep0(rays_o, rays_d, t0, R=256):
    B = rays_o.shape[0]
    grid = (B // R,)
    return pl.pallas_call(
        _prep0_body,
        grid=grid,
        in_specs=[
            pl.BlockSpec((R, 3), lambda i: (i, 0)),
            pl.BlockSpec((R, 3), lambda i: (i, 0)),
            pl.BlockSpec((1, 64), lambda i: (0, 0)),
        ],
        out_specs=[
            pl.BlockSpec((3, R, 128), lambda i: (0, i, 0)),
            pl.BlockSpec((R, 3), lambda i: (i, 0)),
        ],
        out_shape=[
            jax.ShapeDtypeStruct((3, B, 128), jnp.float32),
            jax.ShapeDtypeStruct((B, 3), jnp.float32),
        ],
    )(rays_o, rays_d, t0.reshape(1, 64))


def _stage_body(n_in, n_out, R, K,
                feat_ref, sd_ref, ro_ref, rdn_ref, W1_ref, b1_ref,
                W2_ref, b2_ref, smp_ref, x01_ref, sdn_ref):
    h3 = jnp.zeros((R, n_in, 64), jnp.float32) + b1_ref[...][None]
    for k in range(K):
        h3 = h3 + feat_ref[k][:, :n_in, None] * W1_ref[k][None, None, :]
    h3 = jnp.maximum(h3, 0.0)
    sig_lin = jnp.sum(h3 * W2_ref[...][None], axis=-1) + b2_ref[0, 0]
    sigma = _softplus(sig_lin)
    sd = sd_ref[...]
    weight, _ = _vol_weight(sigma, sd)
    sd_next, ex, ey, ez = _resample_and_project(
        sd, weight, smp_ref[...], ro_ref[...], rdn_ref[...], n_in)
    sdn_ref[...] = sd_next
    pad = jnp.zeros((R, 128 - n_out), jnp.float32)
    x01_ref[0] = jnp.concatenate([ex, pad], -1)
    x01_ref[1] = jnp.concatenate([ey, pad], -1)
    x01_ref[2] = jnp.concatenate([ez, pad], -1)


def _stage(feats, sd_prev, rays_o, rdn, W1, b1, W2, b2, smp, n_in, n_out,
           R=128):
    B = rays_o.shape[0]
    K = feats.shape[0]
    grid = (B // R,)
    body = functools.partial(_stage_body, n_in, n_out, R, K)
    return pl.pallas_call(
        body,
        grid=grid,
        in_specs=[
            pl.BlockSpec((K, R, 128), lambda i: (0, i, 0)),
            pl.BlockSpec((R, n_in + 1), lambda i: (i, 0)),
            pl.BlockSpec((R, 3), lambda i: (i, 0)),
            pl.BlockSpec((R, 3), lambda i: (i, 0)),
            pl.BlockSpec((K, 64), lambda i: (0, 0)),
            pl.BlockSpec((1, 64), lambda i: (0, 0)),
            pl.BlockSpec((1, 64), lambda i: (0, 0)),
            pl.BlockSpec((1, 1), lambda i: (0, 0)),
            pl.BlockSpec((1, n_out), lambda i: (0, 0)),
        ],
        out_specs=[
            pl.BlockSpec((3, R, 128), lambda i: (0, i, 0)),
            pl.BlockSpec((R, n_out + 1), lambda i: (i, 0)),
        ],
        out_shape=[
            jax.ShapeDtypeStruct((3, B, 128), jnp.float32),
            jax.ShapeDtypeStruct((B, n_out + 1), jnp.float32),
        ],
    )(feats, sd_prev, rays_o, rdn, W1, b1.reshape(1, 64),
      W2.reshape(1, 64), b2.reshape(1, 1), smp.reshape(1, n_out))


def _render_body(R, K, feat_ref, sd_ref, rdn_ref, W1_ref, b1_ref, Wsig_ref,
                 bsig_ref, Wb_ref, W2a_ref, W2b_ref, b2_ref, Wrgb_ref,
                 brgb_ref, comp_ref, w_ref, op_ref, dep_ref):
    n = 32
    h3 = jnp.zeros((R, n, 64), jnp.float32) + b1_ref[...][None]
    for k in range(K):
        h3 = h3 + feat_ref[k][:, :n, None] * W1_ref[k][None, None, :]
    h3 = jnp.maximum(h3, 0.0)
    h = h3.reshape(R * n, 64)
    sig_lin = jnp.sum(h3 * Wsig_ref[...][None], axis=-1) + bsig_ref[0, 0]
    sigma = _softplus(sig_lin)

    v = rdn_ref[...]
    embs = [v]
    for i in range(4):
        embs.append(jnp.sin(v * (2.0 ** i)))
        embs.append(jnp.cos(v * (2.0 ** i)))
    vemb = jnp.concatenate(embs, axis=-1)
    vw = jnp.dot(vemb, W2b_ref[...], preferred_element_type=jnp.float32)
    wbw = jnp.dot(Wb_ref[...], W2a_ref[...],
                  preferred_element_type=jnp.float32)
    h2l = jnp.dot(h, wbw, preferred_element_type=jnp.float32)
    h2 = jnp.maximum(
        h2l.reshape(R, n, 64) + vw[:, None, :] + b2_ref[...][None], 0.0)
    rgb_l = jnp.dot(h2.reshape(R * n, 64), Wrgb_ref[...],
                    preferred_element_type=jnp.float32) + brgb_ref[...]
    rgb = jax.nn.sigmoid(rgb_l).reshape(R, n, 3)

    sd = sd_ref[...]
    w, td = _vol_weight(sigma, sd)
    opacity = jnp.sum(w, axis=-1, keepdims=True)
    comp = jnp.sum(w[:, :, None] * rgb, axis=1) + (1.0 - opacity)
    tmid = (td[:, 1:] + td[:, :-1]) / 2.0
    depth = jnp.sum(w * tmid, axis=-1, keepdims=True)
    comp_ref[...] = comp
    w_ref[...] = w
    op_ref[...] = opacity
    dep_ref[...] = depth


def _render(feats, sd, rdn, p, R=128):
    B = sd.shape[0]
    n = 32
    K = feats.shape[0]
    grid = (B // R,)
    body = functools.partial(_render_body, R, K)
    return pl.pallas_call(
        body,
        grid=grid,
        in_specs=[
            pl.BlockSpec((K, R, 128), lambda i: (0, i, 0)),
            pl.BlockSpec((R, n + 1), lambda i: (i, 0)),
            pl.BlockSpec((R, 3), lambda i: (i, 0)),
            pl.BlockSpec((K, 64), lambda i: (0, 0)),
            pl.BlockSpec((1, 64), lambda i: (0, 0)),
            pl.BlockSpec((1, 64), lambda i: (0, 0)),
            pl.BlockSpec((1, 1), lambda i: (0, 0)),
            pl.BlockSpec((64, 15), lambda i: (0, 0)),
            pl.BlockSpec((15, 64), lambda i: (0, 0)),
            pl.BlockSpec((27, 64), lambda i: (0, 0)),
            pl.BlockSpec((1, 64), lambda i: (0, 0)),
            pl.BlockSpec((64, 3), lambda i: (0, 0)),
            pl.BlockSpec((1, 3), lambda i: (0, 0)),
        ],
        out_specs=[
            pl.BlockSpec((R, 3), lambda i: (i, 0)),
            pl.BlockSpec((R, n), lambda i: (i, 0)),
            pl.BlockSpec((R, 1), lambda i: (i, 0)),
            pl.BlockSpec((R, 1), lambda i: (i, 0)),
        ],
        out_shape=[
            jax.ShapeDtypeStruct((B, 3), jnp.float32),
            jax.ShapeDtypeStruct((B, n), jnp.float32),
            jax.ShapeDtypeStruct((B, 1), jnp.float32),
            jax.ShapeDtypeStruct((B, 1), jnp.float32),
        ],
    )(feats, sd, rdn,
      p['rend_W1'], p['rend_b1'].reshape(1, 64),
      p['rend_Wsig'].reshape(1, 64), p['rend_bsig'].reshape(1, 1),
      p['rend_Wb'], p['rend_W2'][:15], p['rend_W2'][15:],
      p['rend_b2'].reshape(1, 64), p['rend_Wrgb'],
      p['rend_brgb'].reshape(1, 3))


def _make_encode(B, n, L, D, base, mx):
    res = _res_list(L, base, mx)
    K = L * D
    mesh = plsc.VectorSubcoreMesh(core_axis_name="c", subcore_axis_name="s")
    rays_pw = B // _NW
    nch = rays_pw // 8
    nvr = n // _LANES
    nv = 8 * nvr
    nsub = nv // 8

    def body(x01, tab, out, xv, yv, zv, fxv, fyv, fzv,
             hxv, hyv, hzv, idxv, wv, valv, featv, semg):
        wid = lax.axis_index("s") * _NC + lax.axis_index("c")
        ray0 = wid * rays_pw

        def chunk(ci, carry):
            b0 = ray0 + ci * 8
            pltpu.sync_copy(x01.at[0, pl.ds(b0, 8), :], xv)
            pltpu.sync_copy(x01.at[1, pl.ds(b0, 8), :], yv)
            pltpu.sync_copy(x01.at[2, pl.ds(b0, 8), :], zv)
            for l in range(L):
                resf = jnp.float32(res[l])
                off = l * _T * D

                def prep(i, c2):
                    r = lax.shift_right_logical(i, _SHV[nvr])
                    j = i & (nvr - 1)
                    s = pl.ds(j * _LANES, _LANES)
                    px = xv[r, s] * resf
                    py = yv[r, s] * resf
                    pz = zv[r, s] * resf
                    ix = px.astype(jnp.int32)
                    iy = py.astype(jnp.int32)
                    iz = pz.astype(jnp.int32)
                    fxv[r, s] = px - ix.astype(jnp.float32)
                    fyv[r, s] = py - iy.astype(jnp.float32)
                    fzv[r, s] = pz - iz.astype(jnp.float32)
                    hxv[r, s] = ix
                    hyv[r, s] = iy * _P2
                    hzv[r, s] = iz * _P3
                    return c2
                lax.fori_loop(0, nv, prep, 0)

                def fire(c, c2):
                    cx = lax.shift_right_logical(c, 2) & 1
                    cy = lax.shift_right_logical(c, 1) & 1
                    cz = c & 1
                    cxf = cx.astype(jnp.float32)
                    cyf = cy.astype(jnp.float32)
                    czf = cz.astype(jnp.float32)

                    def fv(i, c3):
                        r = lax.shift_right_logical(i, _SHV[nvr])
                        j = i & (nvr - 1)
                        s = pl.ds(j * _LANES, _LANES)
                        sub = lax.shift_right_logical(i, 3)
                        so = pl.ds((i & 7) * _LANES, _LANES)
                        hx = hxv[r, s] + cx
                        hy = hyv[r, s] + cy * _P2
                        hz = hzv[r, s] + cz * _P3
                        hh = ((hx ^ hy) ^ hz) & _MASK
                        if D == 1:
                            idxv[c, sub, so] = hh + off
                        else:
                            for d in range(D):
                                idxv[c * D + d, sub, so] = hh * D + off + d
                        fx = fxv[r, s]
                        fy = fyv[r, s]
                        fz = fzv[r, s]
                        wx = cxf * fx + (1.0 - cxf) * (1.0 - fx)
                        wy = cyf * fy + (1.0 - cyf) * (1.0 - fy)
                        wz = czf * fz + (1.0 - czf) * (1.0 - fz)
                        wv[c, sub, so] = wx * wy * wz
                        return c3
                    lax.fori_loop(0, nv, fv, 0)
                    for d in range(D):
                        jrow = c * D + d
                        for srow in range(nsub):
                            pltpu.async_copy(
                                tab.at[idxv.at[jrow, srow]],
                                valv.at[jrow, srow], semg)
                    return c2
                lax.fori_loop(0, 8, fire, 0)

                def drain(c, c2):
                    pltpu.make_async_copy(
                        tab.at[pl.ds(0, 128)], valv.at[0, 0], semg).wait()
                    return c2
                lax.fori_loop(0, 8 * D * nsub, drain, 0)

                def acc(i, c2):
                    r = lax.shift_right_logical(i, _SHV[nvr])
                    j = i & (nvr - 1)
                    s = pl.ds(j * _LANES, _LANES)
                    sub = lax.shift_right_logical(i, 3)
                    so = pl.ds((i & 7) * _LANES, _LANES)
                    if D == 1:
                        def inner(c, a):
                            return a + wv[c, sub, so] * valv[c, sub, so]
                        a = lax.fori_loop(
                            0, 8, inner, jnp.zeros((16,), jnp.float32))
                        featv[l, r, s] = a
                    else:
                        def inner(c, a):
                            w = wv[c, sub, so]
                            return (a[0] + w * valv[c * D, sub, so],
                                    a[1] + w * valv[c * D + 1, sub, so])
                        a = lax.fori_loop(
                            0, 8, inner,
                            (jnp.zeros((16,), jnp.float32),
                             jnp.zeros((16,), jnp.float32)))
                        featv[l * D, r, s] = a[0]
                        featv[l * D + 1, r, s] = a[1]
                    return c2
                lax.fori_loop(0, nv, acc, 0)
            pltpu.sync_copy(featv, out.at[:, pl.ds(b0, 8), :])
            return carry

        lax.fori_loop(0, nch, chunk, 0)

    kern = pl.kernel(
        body,
        out_type=jax.ShapeDtypeStruct((K, B, 128), jnp.float32),
        mesh=mesh,
        scratch_types=[
            pltpu.VMEM((8, 128), jnp.float32),
            pltpu.VMEM((8, 128), jnp.float32),
            pltpu.VMEM((8, 128), jnp.float32),
            pltpu.VMEM((8, 128), jnp.float32),
            pltpu.VMEM((8, 128), jnp.float32),
            pltpu.VMEM((8, 128), jnp.float32),
            pltpu.VMEM((8, 128), jnp.int32),
            pltpu.VMEM((8, 128), jnp.int32),
            pltpu.VMEM((8, 128), jnp.int32),
            pltpu.VMEM((8 * D, nsub, 128), jnp.int32),
            pltpu.VMEM((8, nsub, 128), jnp.float32),
            pltpu.VMEM((8 * D, nsub, 128), jnp.float32),
            pltpu.VMEM((K, 8, 128), jnp.float32),
            pltpu.SemaphoreType.DMA,
        ],
    )
    return kern


_SHV = {2: 1, 4: 2, 8: 3}


def _encode(x01, table, L, D, base, mx, n):
    B = x01.shape[1]
    tab = table.reshape(L * _T * D)
    return _make_encode(B, n, L, D, base, mx)(x01, tab)


def kernel(rays_o, rays_d, params):
    B = rays_d.shape[0]

    s0 = _sample_positions(64, jax.random.key(100))
    s1 = _sample_positions(64, jax.random.key(101))
    s2 = _sample_positions(32, jax.random.key(999))

    new0 = s0
    mid0 = (new0[1:] + new0[:-1]) / 2.0
    first0 = jnp.maximum(2.0 * new0[:1] - mid0[:1], 0.0)
    last0 = jnp.minimum(2.0 * new0[-1:] - mid0[-1:], 1.0)
    sd0 = jnp.concatenate([first0, mid0, last0], axis=-1)
    t0 = _NEAR * (1.0 - new0) + _FAR * new0

    x01_0, rdn = _prep0(rays_o, rays_d, t0)

    feats0 = _encode(x01_0, params['prop_table_0'], 8, 1, 16, 512, 64)
    x01_1, sd1 = _stage(
        feats0, jnp.broadcast_to(sd0, (B, 65)), rays_o, rdn,
        params['prop_W1_0'], params['prop_b1_0'],
        params['prop_W2_0'], params['prop_b2_0'], s1, 64, 64)

    feats1 = _encode(x01_1, params['prop_table_1'], 8, 1, 16, 512, 64)
    x01_2, sd2 = _stage(
        feats1, sd1, rays_o, rdn,
        params['prop_W1_1'], params['prop_b1_1'],
        params['prop_W2_1'], params['prop_b2_1'], s2, 64, 32)

    feats2 = _encode(x01_2, params['render_table'], 16, 2, 16, 2048, 32)
    comp, w, opacity, depth = _render(feats2, sd2, rdn, params)

    return (comp.reshape(B, 1, 1, 3), w.reshape(B, 1, 1, 32),
            opacity.reshape(B, 1, 1), depth.reshape(B, 1, 1))


# speedup vs baseline: 2.4464x; 2.3653x over previous
import functools

import numpy as np
import jax
import jax.numpy as jnp
from jax import lax
from jax.experimental import pallas as pl
from jax.experimental.pallas import tpu as pltpu
from jax.experimental.pallas import tpu_sc as plsc

_NEAR = 2.0
_FAR = 6.0
_T = 2 ** 19
_MASK = _T - 1
_P2 = int(np.int32(np.uint32(2654435761)))
_P3 = int(np.int32(np.uint32(805459861)))
_NC, _NS, _LANES = 2, 16, 16
_NW = _NC * _NS


def _res_list(L, base, mx):
    growth = np.exp((np.log(mx) - np.log(base)) / max(L - 1, 1))
    return [int(np.floor(base * growth ** l)) for l in range(L)]


def _sample_positions(n, key):
    eps = float(jnp.finfo(jnp.float32).eps)
    inter_len = eps + (1.0 - eps) / n
    pos = jnp.linspace(0.0, 1.0 - inter_len, n)
    max_jitter = (1.0 - inter_len) / (n - 1) - eps
    shift = jax.random.uniform(key, (n,)) * max_jitter
    return pos + shift


def _cumsum_last(x):
    n = x.shape[-1]
    k = 1
    while k < n:
        x = x + jnp.concatenate(
            [jnp.zeros_like(x[..., :k]), x[..., :-k]], axis=-1)
        k *= 2
    return x


def _softplus(x):
    return jnp.maximum(x, 0.0) + jnp.log1p(jnp.exp(-jnp.abs(x)))


def _contract_to_unit(x, y, z):
    r = jnp.sqrt(x * x + y * y + z * z)
    rs = jnp.maximum(r, 1e-9)
    scale = jnp.where(r <= 1.0, 1.0, (2.0 - 1.0 / rs) / rs)
    def enc(c):
        return jnp.clip((c * scale + 2.0) / 4.0, 0.0, 1.0 - 1e-6)
    return enc(x), enc(y), enc(z)


def _vol_weight(sigma, sd):
    td = _NEAR * (1.0 - sd) + _FAR * sd
    delta = td[:, 1:] - td[:, :-1]
    dd = sigma * delta
    alpha = 1.0 - jnp.exp(-dd)
    csum = _cumsum_last(dd[:, :-1])
    trans = jnp.exp(-jnp.concatenate(
        [jnp.zeros_like(dd[:, :1]), csum], axis=-1))
    return alpha * trans, td


def _invert_cdf_block(smp, sd, cw, n_in):
    R = sd.shape[0]
    big = jnp.float32(np.inf)
    fp0 = jnp.full((R, smp.shape[1]), -big)
    fp1 = jnp.full((R, smp.shape[1]), big)
    xp0 = jnp.full((R, smp.shape[1]), -big)
    xp1 = jnp.full((R, smp.shape[1]), big)
    for e in range(n_in + 1):
        if e == 0:
            cdf_e = jnp.zeros((R, 1), jnp.float32)
        elif e == n_in:
            cdf_e = jnp.ones((R, 1), jnp.float32)
        else:
            cdf_e = cw[:, e - 1:e]
        sd_e = sd[:, e:e + 1]
        mask = smp >= cdf_e
        fp0 = jnp.maximum(fp0, jnp.where(mask, sd_e, -big))
        fp1 = jnp.minimum(fp1, jnp.where(mask, big, sd_e))
        xp0 = jnp.maximum(xp0, jnp.where(mask, cdf_e, -big))
        xp1 = jnp.minimum(xp1, jnp.where(mask, big, cdf_e))
    raw = (smp - xp0) / (xp1 - xp0)
    off = jnp.clip(jnp.where(raw != raw, 0.0, raw), 0.0, 1.0)
    return fp0 + off * (fp1 - fp0)


def _resample_and_project(sd, weight, smp, ro, rdn, n_in):
    logits = jnp.where(sd[:, 1:] > sd[:, :-1],
                       jnp.log(jnp.maximum(weight, 1e-30)),
                       -jnp.float32(np.inf))
    m = jnp.max(logits, axis=-1, keepdims=True)
    une = jnp.exp(logits - m)
    wsm = une / jnp.sum(une, axis=-1, keepdims=True)
    cw = jnp.minimum(_cumsum_last(wsm[:, :-1]), 1.0)
    new = _invert_cdf_block(smp, sd, cw, n_in)
    mid = (new[:, 1:] + new[:, :-1]) / 2.0
    first = jnp.maximum(2.0 * new[:, :1] - mid[:, :1], 0.0)
    last = jnp.minimum(2.0 * new[:, -1:] - mid[:, -1:], 1.0)
    sd_next = jnp.concatenate([first, mid, last], axis=-1)
    t = _NEAR * (1.0 - new) + _FAR * new
    x = ro[:, 0:1] + rdn[:, 0:1] * t
    y = ro[:, 1:2] + rdn[:, 1:2] * t
    z = ro[:, 2:3] + rdn[:, 2:3] * t
    ex, ey, ez = _contract_to_unit(x, y, z)
    return sd_next, ex, ey, ez


def _prep0_body(ro_ref, rd_ref, t0_ref, x01_ref, rdn_ref):
    rd = rd_ref[...]
    nrm = jnp.sqrt(jnp.sum(rd * rd, axis=-1, keepdims=True))
    rdn = rd / jnp.maximum(nrm, 1e-9)
    rdn_ref[...] = rdn
    ro = ro_ref[...]
    t = t0_ref[...]
    x = ro[:, 0:1] + rdn[:, 0:1] * t
    y = ro[:, 1:2] + rdn[:, 1:2] * t
    z = ro[:, 2:3] + rdn[:, 2:3] * t
    ex, ey, ez = _contract_to_unit(x, y, z)
    pad = jnp.zeros((ex.shape[0], 128 - ex.shape[1]), jnp.float32)
    x01_ref[0] = jnp.concatenate([ex, pad], -1)
    x01_ref[1] = jnp.concatenate([ey, pad], -1)
    x01_ref[2] = jnp.concatenate([ez, pad], -1)


def _prep0(rays_o, rays_d, t0, R=256):
    B = rays_o.shape[0]
    grid = (B // R,)
    return pl.pallas_call(
        _prep0_body,
        grid=grid,
        in_specs=[
            pl.BlockSpec((R, 3), lambda i: (i, 0)),
            pl.BlockSpec((R, 3), lambda i: (i, 0)),
            pl.BlockSpec((1, 64), lambda i: (0, 0)),
        ],
        out_specs=[
            pl.BlockSpec((3, R, 128), lambda i: (0, i, 0)),
            pl.BlockSpec((R, 3), lambda i: (i, 0)),
        ],
        out_shape=[
            jax.ShapeDtypeStruct((3, B, 128), jnp.float32),
            jax.ShapeDtypeStruct((B, 3), jnp.float32),
        ],
    )(rays_o, rays_d, t0.reshape(1, 64))


def _stage_body(n_in, n_out, R, K,
                feat_ref, sd_ref, ro_ref, rdn_ref, W1_ref, b1_ref,
                W2_ref, b2_ref, smp_ref, x01_ref, sdn_ref):
    h3 = jnp.zeros((R, n_in, 64), jnp.float32) + b1_ref[...][None]
    for k in range(K):
        h3 = h3 + feat_ref[k][:, :n_in, None] * W1_ref[k][None, None, :]
    h3 = jnp.maximum(h3, 0.0)
    sig_lin = jnp.sum(h3 * W2_ref[...][None], axis=-1) + b2_ref[0, 0]
    sigma = _softplus(sig_lin)
    sd = sd_ref[...]
    weight, _ = _vol_weight(sigma, sd)
    sd_next, ex, ey, ez = _resample_and_project(
        sd, weight, smp_ref[...], ro_ref[...], rdn_ref[...], n_in)
    sdn_ref[...] = sd_next
    pad = jnp.zeros((R, 128 - n_out), jnp.float32)
    x01_ref[0] = jnp.concatenate([ex, pad], -1)
    x01_ref[1] = jnp.concatenate([ey, pad], -1)
    x01_ref[2] = jnp.concatenate([ez, pad], -1)


def _stage(feats, sd_prev, rays_o, rdn, W1, b1, W2, b2, smp, n_in, n_out,
           R=128):
    B = rays_o.shape[0]
    K = feats.shape[0]
    grid = (B // R,)
    body = functools.partial(_stage_body, n_in, n_out, R, K)
    return pl.pallas_call(
        body,
        grid=grid,
        in_specs=[
            pl.BlockSpec((K, R, 128), lambda i: (0, i, 0)),
            pl.BlockSpec((R, n_in + 1), lambda i: (i, 0)),
            pl.BlockSpec((R, 3), lambda i: (i, 0)),
            pl.BlockSpec((R, 3), lambda i: (i, 0)),
            pl.BlockSpec((K, 64), lambda i: (0, 0)),
            pl.BlockSpec((1, 64), lambda i: (0, 0)),
            pl.BlockSpec((1, 64), lambda i: (0, 0)),
            pl.BlockSpec((1, 1), lambda i: (0, 0)),
            pl.BlockSpec((1, n_out), lambda i: (0, 0)),
        ],
        out_specs=[
            pl.BlockSpec((3, R, 128), lambda i: (0, i, 0)),
            pl.BlockSpec((R, n_out + 1), lambda i: (i, 0)),
        ],
        out_shape=[
            jax.ShapeDtypeStruct((3, B, 128), jnp.float32),
            jax.ShapeDtypeStruct((B, n_out + 1), jnp.float32),
        ],
    )(feats, sd_prev, rays_o, rdn, W1, b1.reshape(1, 64),
      W2.reshape(1, 64), b2.reshape(1, 1), smp.reshape(1, n_out))


def _render_body(R, K, feat_ref, sd_ref, rdn_ref, W1_ref, b1_ref, Wsig_ref,
                 bsig_ref, Wb_ref, W2a_ref, W2b_ref, b2_ref, Wrgb_ref,
                 brgb_ref, comp_ref, w_ref, op_ref, dep_ref):
    n = 32
    h3 = jnp.zeros((R, n, 64), jnp.float32) + b1_ref[...][None]
    for k in range(K):
        h3 = h3 + feat_ref[k][:, :n, None] * W1_ref[k][None, None, :]
    h3 = jnp.maximum(h3, 0.0)
    h = h3.reshape(R * n, 64)
    sig_lin = jnp.sum(h3 * Wsig_ref[...][None], axis=-1) + bsig_ref[0, 0]
    sigma = _softplus(sig_lin)

    v = rdn_ref[...]
    embs = [v]
    for i in range(4):
        embs.append(jnp.sin(v * (2.0 ** i)))
        embs.append(jnp.cos(v * (2.0 ** i)))
    vemb = jnp.concatenate(embs, axis=-1)
    vw = jnp.dot(vemb, W2b_ref[...], preferred_element_type=jnp.float32)
    wbw = jnp.dot(Wb_ref[...], W2a_ref[...],
                  preferred_element_type=jnp.float32)
    h2l = jnp.dot(h, wbw, preferred_element_type=jnp.float32)
    h2 = jnp.maximum(
        h2l.reshape(R, n, 64) + vw[:, None, :] + b2_ref[...][None], 0.0)
    rgb_l = jnp.dot(h2.reshape(R * n, 64), Wrgb_ref[...],
                    preferred_element_type=jnp.float32) + brgb_ref[...]
    rgb = jax.nn.sigmoid(rgb_l).reshape(R, n, 3)

    sd = sd_ref[...]
    w, td = _vol_weight(sigma, sd)
    opacity = jnp.sum(w, axis=-1, keepdims=True)
    comp = jnp.sum(w[:, :, None] * rgb, axis=1) + (1.0 - opacity)
    tmid = (td[:, 1:] + td[:, :-1]) / 2.0
    depth = jnp.sum(w * tmid, axis=-1, keepdims=True)
    comp_ref[...] = comp
    w_ref[...] = w
    op_ref[...] = opacity
    dep_ref[...] = depth


def _render(feats, sd, rdn, p, R=128):
    B = sd.shape[0]
    n = 32
    K = feats.shape[0]
    grid = (B // R,)
    body = functools.partial(_render_body, R, K)
    return pl.pallas_call(
        body,
        grid=grid,
        in_specs=[
            pl.BlockSpec((K, R, 128), lambda i: (0, i, 0)),
            pl.BlockSpec((R, n + 1), lambda i: (i, 0)),
            pl.BlockSpec((R, 3), lambda i: (i, 0)),
            pl.BlockSpec((K, 64), lambda i: (0, 0)),
            pl.BlockSpec((1, 64), lambda i: (0, 0)),
            pl.BlockSpec((1, 64), lambda i: (0, 0)),
            pl.BlockSpec((1, 1), lambda i: (0, 0)),
            pl.BlockSpec((64, 15), lambda i: (0, 0)),
            pl.BlockSpec((15, 64), lambda i: (0, 0)),
            pl.BlockSpec((27, 64), lambda i: (0, 0)),
            pl.BlockSpec((1, 64), lambda i: (0, 0)),
            pl.BlockSpec((64, 3), lambda i: (0, 0)),
            pl.BlockSpec((1, 3), lambda i: (0, 0)),
        ],
        out_specs=[
            pl.BlockSpec((R, 3), lambda i: (i, 0)),
            pl.BlockSpec((R, n), lambda i: (i, 0)),
            pl.BlockSpec((R, 1), lambda i: (i, 0)),
            pl.BlockSpec((R, 1), lambda i: (i, 0)),
        ],
        out_shape=[
            jax.ShapeDtypeStruct((B, 3), jnp.float32),
            jax.ShapeDtypeStruct((B, n), jnp.float32),
            jax.ShapeDtypeStruct((B, 1), jnp.float32),
            jax.ShapeDtypeStruct((B, 1), jnp.float32),
        ],
    )(feats, sd, rdn,
      p['rend_W1'], p['rend_b1'].reshape(1, 64),
      p['rend_Wsig'].reshape(1, 64), p['rend_bsig'].reshape(1, 1),
      p['rend_Wb'], p['rend_W2'][:15], p['rend_W2'][15:],
      p['rend_b2'].reshape(1, 64), p['rend_Wrgb'],
      p['rend_brgb'].reshape(1, 3))


def _make_encode(B, n, L, D, base, mx):
    res = _res_list(L, base, mx)
    K = L * D
    mesh = plsc.VectorSubcoreMesh(core_axis_name="c", subcore_axis_name="s")
    rays_pw = B // _NW
    nch = rays_pw // 8
    nvr = n // _LANES
    nv = 8 * nvr
    nsub = nv // 8

    def body(x01, tab, out, xv, yv, zv, fxv, fyv, fzv,
             hxv, hyv, hzv, idxv, wv, valv, featv, semg):
        wid = lax.axis_index("s") * _NC + lax.axis_index("c")
        ray0 = wid * rays_pw

        def chunk(ci, carry):
            b0 = ray0 + ci * 8
            pltpu.sync_copy(x01.at[0, pl.ds(b0, 8), :], xv)
            pltpu.sync_copy(x01.at[1, pl.ds(b0, 8), :], yv)
            pltpu.sync_copy(x01.at[2, pl.ds(b0, 8), :], zv)
            for l in range(L):
                resf = jnp.float32(res[l])
                off = l * _T * D

                def prep(i, c2):
                    r = lax.shift_right_logical(i, _SHV[nvr])
                    j = i & (nvr - 1)
                    s = pl.ds(j * _LANES, _LANES)
                    px = xv[r, s] * resf
                    py = yv[r, s] * resf
                    pz = zv[r, s] * resf
                    ix = px.astype(jnp.int32)
                    iy = py.astype(jnp.int32)
                    iz = pz.astype(jnp.int32)
                    fxv[r, s] = px - ix.astype(jnp.float32)
                    fyv[r, s] = py - iy.astype(jnp.float32)
                    fzv[r, s] = pz - iz.astype(jnp.float32)
                    hxv[r, s] = ix
                    hyv[r, s] = iy * _P2
                    hzv[r, s] = iz * _P3
                    return c2
                lax.fori_loop(0, nv, prep, 0)

                def fire(c, c2):
                    cx = lax.shift_right_logical(c, 2) & 1
                    cy = lax.shift_right_logical(c, 1) & 1
                    cz = c & 1
                    cxf = cx.astype(jnp.float32)
                    cyf = cy.astype(jnp.float32)
                    czf = cz.astype(jnp.float32)

                    def fv(i, c3):
                        r = lax.shift_right_logical(i, _SHV[nvr])
                        j = i & (nvr - 1)
                        s = pl.ds(j * _LANES, _LANES)
                        sub = lax.shift_right_logical(i, 3)
                        so = pl.ds((i & 7) * _LANES, _LANES)
                        hx = hxv[r, s] + cx
                        hy = hyv[r, s] + cy * _P2
                        hz = hzv[r, s] + cz * _P3
                        hh = ((hx ^ hy) ^ hz) & _MASK
                        if D == 1:
                            idxv[c, sub, so] = hh + off
                        else:
                            bi = off + (hh & ~jnp.int32(127)) * D + (hh & 127)
                            for d in range(D):
                                idxv[c * D + d, sub, so] = bi + d * 128
                        fx = fxv[r, s]
                        fy = fyv[r, s]
                        fz = fzv[r, s]
                        wx = cxf * fx + (1.0 - cxf) * (1.0 - fx)
                        wy = cyf * fy + (1.0 - cyf) * (1.0 - fy)
                        wz = czf * fz + (1.0 - czf) * (1.0 - fz)
                        wv[c, sub, so] = wx * wy * wz
                        return c3
                    lax.fori_loop(0, nv, fv, 0)
                    for d in range(D):
                        jrow = c * D + d
                        for srow in range(nsub):
                            pltpu.async_copy(
                                tab.at[idxv.at[jrow, srow]],
                                valv.at[jrow, srow], semg)
                    return c2
                lax.fori_loop(0, 8, fire, 0)

                def drain(c, c2):
                    pltpu.make_async_copy(
                        tab.at[pl.ds(0, 128)], valv.at[0, 0], semg).wait()
                    return c2
                lax.fori_loop(0, 8 * D * nsub, drain, 0)

                def acc(i, c2):
                    r = lax.shift_right_logical(i, _SHV[nvr])
                    j = i & (nvr - 1)
                    s = pl.ds(j * _LANES, _LANES)
                    sub = lax.shift_right_logical(i, 3)
                    so = pl.ds((i & 7) * _LANES, _LANES)
                    if D == 1:
                        def inner(c, a):
                            return a + wv[c, sub, so] * valv[c, sub, so]
                        a = lax.fori_loop(
                            0, 8, inner, jnp.zeros((16,), jnp.float32))
                        featv[l, r, s] = a
                    else:
                        def inner(c, a):
                            w = wv[c, sub, so]
                            return (a[0] + w * valv[c * D, sub, so],
                                    a[1] + w * valv[c * D + 1, sub, so])
                        a = lax.fori_loop(
                            0, 8, inner,
                            (jnp.zeros((16,), jnp.float32),
                             jnp.zeros((16,), jnp.float32)))
                        featv[l * D, r, s] = a[0]
                        featv[l * D + 1, r, s] = a[1]
                    return c2
                lax.fori_loop(0, nv, acc, 0)
            pltpu.sync_copy(featv, out.at[:, pl.ds(b0, 8), :])
            return carry

        lax.fori_loop(0, nch, chunk, 0)

    kern = pl.kernel(
        body,
        out_type=jax.ShapeDtypeStruct((K, B, 128), jnp.float32),
        mesh=mesh,
        scratch_types=[
            pltpu.VMEM((8, 128), jnp.float32),
            pltpu.VMEM((8, 128), jnp.float32),
            pltpu.VMEM((8, 128), jnp.float32),
            pltpu.VMEM((8, 128), jnp.float32),
            pltpu.VMEM((8, 128), jnp.float32),
            pltpu.VMEM((8, 128), jnp.float32),
            pltpu.VMEM((8, 128), jnp.int32),
            pltpu.VMEM((8, 128), jnp.int32),
            pltpu.VMEM((8, 128), jnp.int32),
            pltpu.VMEM((8 * D, nsub, 128), jnp.int32),
            pltpu.VMEM((8, nsub, 128), jnp.float32),
            pltpu.VMEM((8 * D, nsub, 128), jnp.float32),
            pltpu.VMEM((K, 8, 128), jnp.float32),
            pltpu.SemaphoreType.DMA,
        ],
    )
    return kern


_SHV = {2: 1, 4: 2, 8: 3}


def _encode(x01, table, L, D, base, mx, n):
    B = x01.shape[1]
    if D == 1:
        tab = table.reshape(L * _T)
    else:
        tab = (table.reshape(L, _T // 128, 128, D)
               .transpose(0, 1, 3, 2)
               .reshape(L * _T * D))
    return _make_encode(B, n, L, D, base, mx)(x01, tab)


def kernel(rays_o, rays_d, params):
    B = rays_d.shape[0]

    s0 = _sample_positions(64, jax.random.key(100))
    s1 = _sample_positions(64, jax.random.key(101))
    s2 = _sample_positions(32, jax.random.key(999))

    new0 = s0
    mid0 = (new0[1:] + new0[:-1]) / 2.0
    first0 = jnp.maximum(2.0 * new0[:1] - mid0[:1], 0.0)
    last0 = jnp.minimum(2.0 * new0[-1:] - mid0[-1:], 1.0)
    sd0 = jnp.concatenate([first0, mid0, last0], axis=-1)
    t0 = _NEAR * (1.0 - new0) + _FAR * new0

    x01_0, rdn = _prep0(rays_o, rays_d, t0)

    feats0 = _encode(x01_0, params['prop_table_0'], 8, 1, 16, 512, 64)
    x01_1, sd1 = _stage(
        feats0, jnp.broadcast_to(sd0, (B, 65)), rays_o, rdn,
        params['prop_W1_0'], params['prop_b1_0'],
        params['prop_W2_0'], params['prop_b2_0'], s1, 64, 64)

    feats1 = _encode(x01_1, params['prop_table_1'], 8, 1, 16, 512, 64)
    x01_2, sd2 = _stage(
        feats1, sd1, rays_o, rdn,
        params['prop_W1_1'], params['prop_b1_1'],
        params['prop_W2_1'], params['prop_b2_1'], s2, 64, 32)

    feats2 = _encode(x01_2, params['render_table'], 16, 2, 16, 2048, 32)
    comp, w, opacity, depth = _render(feats2, sd2, rdn, params)

    return (comp.reshape(B, 1, 1, 3), w.reshape(B, 1, 1, 32),
            opacity.reshape(B, 1, 1), depth.reshape(B, 1, 1))
